# baseline (speedup 1.0000x reference)
import jax
import jax.numpy as jnp
from jax import lax
from jax.experimental import pallas as pl
from jax.experimental.pallas import tpu as pltpu

_ROWS_PER_BLOCK = 32
_N = 32768
_LANES = 128
_CHUNKS = _N // _LANES


def _mask_kernel(k_ref, probs_ref, gumbel_ref, out_ref):
    k = k_ref[0]
    p = probs_ref[...]
    u = gumbel_ref[...]

    eps = 1e-20
    inner = -jnp.log(jnp.maximum(u, eps))
    gumbel_noise = -jnp.log(jnp.maximum(inner, eps))
    conf = jnp.log(p + 1e-05) + gumbel_noise

    bits = lax.bitcast_convert_type(conf, jnp.uint32)
    flip = jnp.where(
        (bits >> 31) == jnp.uint32(1),
        jnp.uint32(0xFFFFFFFF),
        jnp.uint32(0x80000000),
    )
    ukey = bits ^ flip

    rows = p.shape[0]
    ones_bf = jnp.ones((_N, 1), jnp.bfloat16)
    one_b = jnp.bfloat16(1)
    zero_b = jnp.bfloat16(0)
    k_f = k.astype(jnp.float32)

    bias = jnp.uint32(0x8000)
    hi = ((ukey >> 16) ^ bias).astype(jnp.int16)
    lo = (ukey ^ bias).astype(jnp.int16)

    def count_lt(arr, cand_i16):
        sel = jnp.where(arr < cand_i16, one_b, zero_b)
        return jax.lax.dot_general(
            sel, ones_bf, (((1,), (0,)), ((), ())),
            preferred_element_type=jnp.float32,
        )

    sb = jnp.int32(0x8000)

    def hi_body(_, carry):
        tpref, bit = carry
        cand = tpref | bit
        cnt = count_lt(hi, (cand ^ sb).astype(jnp.int16))
        tpref = jnp.where(cnt >= k_f, tpref, cand)
        return tpref, bit >> 1

    t0 = jnp.zeros((rows, 1), jnp.int32)
    T_hi, _ = lax.fori_loop(0, 16, hi_body, (t0, jnp.int32(1 << 15)))

    t_hi_i16 = (T_hi ^ sb).astype(jnp.int16)
    c_hi = count_lt(hi, t_hi_i16)
    need_lo = k_f - c_hi
    pm = hi == t_hi_i16
    masked_lo = jnp.where(pm, lo, jnp.int16(32767))

    def lo_body(_, carry):
        tpref, bit = carry
        cand = tpref | bit
        cnt = count_lt(masked_lo, (cand ^ sb).astype(jnp.int16))
        tpref = jnp.where(cnt >= need_lo, tpref, cand)
        return tpref, bit >> 1

    T_lo, _ = lax.fori_loop(0, 16, lo_body, (t0, jnp.int32(1 << 15)))

    T = (T_hi.astype(jnp.uint32) << 16) | T_lo.astype(jnp.uint32)
    lt = ukey < T
    eq = ukey == T
    ltf = jnp.where(lt, 1.0, 0.0)
    c_lt = jax.lax.dot_general(
        ltf, jnp.ones((_N, 1), jnp.float32), (((1,), (0,)), ((), ())),
        preferred_element_type=jnp.float32,
    )
    need = k_f - c_lt

    eqf = jnp.where(eq, 1.0, 0.0)
    e2 = eqf.reshape(rows * _CHUNKS, _LANES)
    li = lax.broadcasted_iota(jnp.int32, (_LANES, _LANES), 0)
    lj = lax.broadcasted_iota(jnp.int32, (_LANES, _LANES), 1)
    lt_incl = jnp.where(li <= lj, 1.0, 0.0)
    intra = jax.lax.dot_general(
        e2, lt_incl, (((1,), (0,)), ((), ())),
        preferred_element_type=jnp.float32,
    )
    totals = jax.lax.dot_general(
        e2, jnp.ones((_LANES, 1), jnp.float32), (((1,), (0,)), ((), ())),
        preferred_element_type=jnp.float32,
    ).reshape(rows, _CHUNKS)
    ci = lax.broadcasted_iota(jnp.int32, (_CHUNKS, _CHUNKS), 0)
    cj = lax.broadcasted_iota(jnp.int32, (_CHUNKS, _CHUNKS), 1)
    slt = jnp.where(ci < cj, 1.0, 0.0)
    offs = jax.lax.dot_general(
        totals, slt, (((1,), (0,)), ((), ())),
        preferred_element_type=jnp.float32,
    )
    rank = intra.reshape(rows, _CHUNKS, _LANES) + offs[:, :, None]
    rank = rank.reshape(rows, _N)

    out_ref[...] = lt | (eq & (rank <= need))


def kernel(probs, gumbel_u, mask_len):
    rows, n = probs.shape
    assert n == _N
    k = jnp.asarray(mask_len, jnp.int32).reshape(1)
    grid = (rows // _ROWS_PER_BLOCK,)
    out = pl.pallas_call(
        _mask_kernel,
        grid=grid,
        in_specs=[
            pl.BlockSpec(memory_space=pltpu.SMEM),
            pl.BlockSpec((_ROWS_PER_BLOCK, _N), lambda i: (i, 0)),
            pl.BlockSpec((_ROWS_PER_BLOCK, _N), lambda i: (i, 0)),
        ],
        out_specs=pl.BlockSpec((_ROWS_PER_BLOCK, _N), lambda i: (i, 0)),
        out_shape=jax.ShapeDtypeStruct((rows, n), jnp.bool_),
    )(k, probs, gumbel_u)
    return out


# speedup vs baseline: 1.0719x; 1.0719x over previous
import jax
import jax.numpy as jnp
from jax import lax
from jax.experimental import pallas as pl
from jax.experimental.pallas import tpu as pltpu

_ROWS_PER_BLOCK = 32
_N = 32768
_LANES = 128
_CHUNKS = _N // _LANES


def _mask_kernel(k_ref, probs_ref, gumbel_ref, out_ref):
    k = k_ref[0]
    p = probs_ref[...]
    u = gumbel_ref[...]

    eps = 1e-20
    inner = -jnp.log(jnp.maximum(u, eps))
    r = (p + 1e-05) / inner
    ukey = lax.bitcast_convert_type(r, jnp.uint32)

    rows = p.shape[0]
    ones_bf = jnp.ones((_N, 1), jnp.bfloat16)
    one_b = jnp.bfloat16(1)
    zero_b = jnp.bfloat16(0)
    k_f = k.astype(jnp.float32)

    bias = jnp.uint32(0x8000)
    hi = (ukey >> 16).astype(jnp.int16)
    lo = (ukey ^ bias).astype(jnp.int16)

    def count_lt(arr, cand_i16):
        sel = jnp.where(arr < cand_i16, one_b, zero_b)
        return jax.lax.dot_general(
            sel, ones_bf, (((1,), (0,)), ((), ())),
            preferred_element_type=jnp.float32,
        )

    sb = jnp.int32(0x8000)

    def hi_body(_, carry):
        tpref, bit = carry
        cand = tpref | bit
        cnt = count_lt(hi, cand.astype(jnp.int16))
        tpref = jnp.where(cnt >= k_f, tpref, cand)
        return tpref, bit >> 1

    t0 = jnp.zeros((rows, 1), jnp.int32)
    T_hi, _ = lax.fori_loop(0, 15, hi_body, (t0, jnp.int32(1 << 14)))

    t_hi_i16 = T_hi.astype(jnp.int16)
    c_hi = count_lt(hi, t_hi_i16)
    need_lo = k_f - c_hi
    pm = hi == t_hi_i16
    masked_lo = jnp.where(pm, lo, jnp.int16(32767))

    def lo_body(_, carry):
        tpref, bit = carry
        cand = tpref | bit
        cnt = count_lt(masked_lo, (cand ^ sb).astype(jnp.int16))
        tpref = jnp.where(cnt >= need_lo, tpref, cand)
        return tpref, bit >> 1

    T_lo, _ = lax.fori_loop(0, 16, lo_body, (t0, jnp.int32(1 << 15)))

    T = (T_hi.astype(jnp.uint32) << 16) | T_lo.astype(jnp.uint32)
    lt = ukey < T
    eq = ukey == T
    ltf = jnp.where(lt, 1.0, 0.0)
    c_lt = jax.lax.dot_general(
        ltf, jnp.ones((_N, 1), jnp.float32), (((1,), (0,)), ((), ())),
        preferred_element_type=jnp.float32,
    )
    need = k_f - c_lt

    eqf = jnp.where(eq, 1.0, 0.0)
    e2 = eqf.reshape(rows * _CHUNKS, _LANES)
    li = lax.broadcasted_iota(jnp.int32, (_LANES, _LANES), 0)
    lj = lax.broadcasted_iota(jnp.int32, (_LANES, _LANES), 1)
    lt_incl = jnp.where(li <= lj, 1.0, 0.0)
    intra = jax.lax.dot_general(
        e2, lt_incl, (((1,), (0,)), ((), ())),
        preferred_element_type=jnp.float32,
    )
    totals = jax.lax.dot_general(
        e2, jnp.ones((_LANES, 1), jnp.float32), (((1,), (0,)), ((), ())),
        preferred_element_type=jnp.float32,
    ).reshape(rows, _CHUNKS)
    ci = lax.broadcasted_iota(jnp.int32, (_CHUNKS, _CHUNKS), 0)
    cj = lax.broadcasted_iota(jnp.int32, (_CHUNKS, _CHUNKS), 1)
    slt = jnp.where(ci < cj, 1.0, 0.0)
    offs = jax.lax.dot_general(
        totals, slt, (((1,), (0,)), ((), ())),
        preferred_element_type=jnp.float32,
    )
    rank = intra.reshape(rows, _CHUNKS, _LANES) + offs[:, :, None]
    rank = rank.reshape(rows, _N)

    out_ref[...] = lt | (eq & (rank <= need))


def kernel(probs, gumbel_u, mask_len):
    rows, n = probs.shape
    assert n == _N
    k = jnp.asarray(mask_len, jnp.int32).reshape(1)
    grid = (rows // _ROWS_PER_BLOCK,)
    out = pl.pallas_call(
        _mask_kernel,
        grid=grid,
        in_specs=[
            pl.BlockSpec(memory_space=pltpu.SMEM),
            pl.BlockSpec((_ROWS_PER_BLOCK, _N), lambda i: (i, 0)),
            pl.BlockSpec((_ROWS_PER_BLOCK, _N), lambda i: (i, 0)),
        ],
        out_specs=pl.BlockSpec((_ROWS_PER_BLOCK, _N), lambda i: (i, 0)),
        out_shape=jax.ShapeDtypeStruct((rows, n), jnp.bool_),
    )(k, probs, gumbel_u)
    return out
